# baseline (speedup 1.0000x reference)
import functools

import jax
import jax.numpy as jnp
from jax import lax
from jax.experimental import pallas as pl
from jax.experimental.pallas import tpu as pltpu
from jax.experimental.pallas import tpu_sc as plsc

VOCAB = 30522
HIDDEN = 1024
MAX_POS = 512
BATCH = 32
SEQ = 512
EPS = 1e-12

N_TOKENS = BATCH * SEQ
NUM_WORKERS = 32
TOK_PER_W = N_TOKENS // NUM_WORKERS
CHUNK = 32
N_CHUNKS = TOK_PER_W // CHUNK


H2 = HIDDEN // 2


def _pack_rows_u32(rows_v, ubuf_v):
    GB = 8

    def row_body(r, carry):
        for j0 in range(0, H2 // 16, GB):
            avs = [lax.bitcast_convert_type(
                rows_v[r, pl.ds(16 * (j0 + k), 16)], jnp.uint32)
                for k in range(GB)]
            bvs = [lax.bitcast_convert_type(
                rows_v[r, pl.ds(H2 + 16 * (j0 + k), 16)], jnp.uint32)
                for k in range(GB)]
            ws = [((avs[k] + 0x8000) >> 16) |
                  ((bvs[k] + 0x8000) & jnp.uint32(0xFFFF0000))
                  for k in range(GB)]
            for k in range(GB):
                ubuf_v[r, pl.ds(16 * (j0 + k), 16)] = ws[k]
        return carry

    lax.fori_loop(0, CHUNK, row_body, 0)


def _sc_gather_body(n_chunks, ids_hbm, table_hbm, out_hbm,
                    idx0, idx1, rows0, rows1, bbuf0, bbuf1,
                    gsem0, gsem1, wsem0, wsem1):
    wid = lax.axis_index("s") * 2 + lax.axis_index("c")
    base = wid * n_chunks
    idx = (idx0, idx1)
    rows = (rows0, rows1)
    bbuf = (bbuf0, bbuf1)
    gsem = (gsem0, gsem1)
    wsem = (wsem0, wsem1)

    def gather_cd(b):
        return pltpu.make_async_copy(table_hbm.at[idx[b]], rows[b], gsem[b])

    def write_cd(c, b):
        return pltpu.make_async_copy(
            bbuf[b], out_hbm.at[pl.ds((base + c) * CHUNK, CHUNK)], wsem[b])

    def start_gather(c, b):
        pltpu.sync_copy(ids_hbm.at[base + c], idx[b])
        gather_cd(b).start()

    start_gather(0, 0)
    if n_chunks > 1:
        start_gather(1, 1)
    for c in range(n_chunks):
        b = c % 2
        gather_cd(b).wait()
        if c + 2 < n_chunks:
            pass
        if c >= 2:
            write_cd(c - 2, b).wait()
        _pack_rows_u32(rows[b], bbuf[b])
        write_cd(c, b).start()
        if c + 2 < n_chunks:
            start_gather(c + 2, b)
    for c in range(max(0, n_chunks - 2), n_chunks):
        write_cd(c, c % 2).wait()


@jax.jit
def _sc_gather(ids2d, table):
    n_rows = ids2d.shape[0]
    n_chunks = n_rows // NUM_WORKERS
    mesh = plsc.VectorSubcoreMesh(core_axis_name="c", subcore_axis_name="s")
    return pl.kernel(
        functools.partial(_sc_gather_body, n_chunks),
        out_type=jax.ShapeDtypeStruct((n_rows * CHUNK, H2), jnp.uint32),
        mesh=mesh,
        scratch_types=[
            pltpu.VMEM((CHUNK,), jnp.int32),
            pltpu.VMEM((CHUNK,), jnp.int32),
            pltpu.VMEM((CHUNK, HIDDEN), jnp.float32),
            pltpu.VMEM((CHUNK, HIDDEN), jnp.float32),
            pltpu.VMEM((CHUNK, H2), jnp.uint32),
            pltpu.VMEM((CHUNK, H2), jnp.uint32),
            pltpu.SemaphoreType.DMA,
            pltpu.SemaphoreType.DMA,
            pltpu.SemaphoreType.DMA,
            pltpu.SemaphoreType.DMA,
        ],
    )(ids2d, table)


TOK_BLK = 512
N_BLKS = N_TOKENS // TOK_BLK


def _tc_ln_body(g_ref, pos_ref, tt_ref, type_ref, gam_ref, bet_ref, out_ref):
    w = g_ref[...]
    xl = lax.bitcast_convert_type(w << 16, jnp.float32)
    xh = lax.bitcast_convert_type(w & jnp.uint32(0xFFFF0000), jnp.float32)
    x = jnp.concatenate([xl, xh], axis=1) + pos_ref[...]
    t0 = type_ref[0:1, :]
    dt = type_ref[1:2, :] - t0
    tt = tt_ref[0, 0, :]
    x = x + t0 + tt[:, None] * dt
    mean = jnp.mean(x, axis=-1, keepdims=True)
    xc = x - mean
    var = jnp.mean(xc * xc, axis=-1, keepdims=True)
    y = xc * lax.rsqrt(var + EPS)
    out_ref[...] = y * gam_ref[...] + bet_ref[...]


SLICE_BATCHES = (4, 6, 10, 12)
SLICE_START = tuple(sum(SLICE_BATCHES[:s]) for s in range(len(SLICE_BATCHES)))


def _make_tc_ln(s):
    aliased = s > 0
    n_b = SLICE_BATCHES[s]

    def body(*refs):
        if aliased:
            g_ref, pos_ref, tt_ref, type_ref, gam_ref, bet_ref, _, out_ref = refs
        else:
            g_ref, pos_ref, tt_ref, type_ref, gam_ref, bet_ref, out_ref = refs
        _tc_ln_body(g_ref, pos_ref, tt_ref, type_ref, gam_ref, bet_ref,
                    out_ref)

    blk0 = SLICE_START[s]
    in_specs = [
        pl.BlockSpec((TOK_BLK, H2), lambda j: (j, 0)),
        pl.BlockSpec((TOK_BLK, HIDDEN), lambda j: (0, 0)),
        pl.BlockSpec((1, 1, TOK_BLK), lambda j: (j, 0, 0)),
        pl.BlockSpec((2, HIDDEN), lambda j: (0, 0)),
        pl.BlockSpec((1, HIDDEN), lambda j: (0, 0)),
        pl.BlockSpec((1, HIDDEN), lambda j: (0, 0)),
    ]
    if aliased:
        in_specs.append(pl.BlockSpec(memory_space=pl.ANY))
    return pl.pallas_call(
        body,
        grid=(n_b,),
        in_specs=in_specs,
        out_specs=pl.BlockSpec((TOK_BLK, HIDDEN), lambda j: (blk0 + j, 0)),
        out_shape=jax.ShapeDtypeStruct((N_TOKENS, HIDDEN), jnp.float32),
        input_output_aliases={6: 0} if aliased else {},
    )


@jax.jit
def _pipeline(ids2d, word_table, pos_table, ttf, type_table, gamma, beta):
    rows_per_batch = SEQ // CHUNK
    gs = [
        _sc_gather(
            lax.slice_in_dim(ids2d, SLICE_START[s] * rows_per_batch,
                             (SLICE_START[s] + SLICE_BATCHES[s]) * rows_per_batch,
                             axis=0),
            word_table)
        for s in range(len(SLICE_BATCHES))
    ]
    buf = None
    for s in range(len(SLICE_BATCHES)):
        tt_s = lax.slice_in_dim(ttf, SLICE_START[s],
                                SLICE_START[s] + SLICE_BATCHES[s], axis=0)
        args = (gs[s], pos_table, tt_s, type_table, gamma, beta)
        buf = _make_tc_ln(s)(*(args if buf is None else args + (buf,)))
    return buf


def kernel(input_ids, token_type_ids, word_table, pos_table, type_table,
           ln_gamma, ln_beta):
    ids2d = input_ids.reshape(N_TOKENS // CHUNK, CHUNK).astype(jnp.int32)
    ttf = token_type_ids.reshape(N_BLKS, 1, TOK_BLK).astype(jnp.float32)
    out = _pipeline(ids2d, word_table, pos_table, ttf, type_table,
                    ln_gamma.reshape(1, HIDDEN), ln_beta.reshape(1, HIDDEN))
    return out.reshape(BATCH, SEQ, HIDDEN)


# speedup vs baseline: 1.0768x; 1.0768x over previous
import functools

import jax
import jax.numpy as jnp
from jax import lax
from jax.experimental import pallas as pl
from jax.experimental.pallas import tpu as pltpu
from jax.experimental.pallas import tpu_sc as plsc

VOCAB = 30522
HIDDEN = 1024
MAX_POS = 512
BATCH = 32
SEQ = 512
EPS = 1e-12

N_TOKENS = BATCH * SEQ
NUM_WORKERS = 32
TOK_PER_W = N_TOKENS // NUM_WORKERS
CHUNK = 32
N_CHUNKS = TOK_PER_W // CHUNK


H2 = HIDDEN // 2


def _pack_rows_u32(rows_v, ubuf_v):
    GB = 8

    def row_body(r, carry):
        for j0 in range(0, H2 // 16, GB):
            avs = [lax.bitcast_convert_type(
                rows_v[r, pl.ds(16 * (j0 + k), 16)], jnp.uint32)
                for k in range(GB)]
            bvs = [lax.bitcast_convert_type(
                rows_v[r, pl.ds(H2 + 16 * (j0 + k), 16)], jnp.uint32)
                for k in range(GB)]
            ws = [(avs[k] >> 16) | (bvs[k] & jnp.uint32(0xFFFF0000))
                  for k in range(GB)]
            for k in range(GB):
                ubuf_v[r, pl.ds(16 * (j0 + k), 16)] = ws[k]
        return carry

    lax.fori_loop(0, CHUNK, row_body, 0)


def _sc_gather_body(n_chunks, ids_hbm, table_hbm, out_hbm,
                    idx0, idx1, rows0, rows1, bbuf0, bbuf1,
                    gsem0, gsem1, wsem0, wsem1):
    wid = lax.axis_index("s") * 2 + lax.axis_index("c")
    base = wid * n_chunks
    idx = (idx0, idx1)
    rows = (rows0, rows1)
    bbuf = (bbuf0, bbuf1)
    gsem = (gsem0, gsem1)
    wsem = (wsem0, wsem1)

    def gather_cd(b):
        return pltpu.make_async_copy(table_hbm.at[idx[b]], rows[b], gsem[b])

    def write_cd(c, b):
        return pltpu.make_async_copy(
            bbuf[b], out_hbm.at[pl.ds((base + c) * CHUNK, CHUNK)], wsem[b])

    def start_gather(c, b):
        pltpu.sync_copy(ids_hbm.at[base + c], idx[b])
        gather_cd(b).start()

    start_gather(0, 0)
    if n_chunks > 1:
        start_gather(1, 1)
    for c in range(n_chunks):
        b = c % 2
        gather_cd(b).wait()
        if c + 2 < n_chunks:
            pass
        if c >= 2:
            write_cd(c - 2, b).wait()
        _pack_rows_u32(rows[b], bbuf[b])
        write_cd(c, b).start()
        if c + 2 < n_chunks:
            start_gather(c + 2, b)
    for c in range(max(0, n_chunks - 2), n_chunks):
        write_cd(c, c % 2).wait()


@jax.jit
def _sc_gather(ids2d, table):
    n_rows = ids2d.shape[0]
    n_chunks = n_rows // NUM_WORKERS
    mesh = plsc.VectorSubcoreMesh(core_axis_name="c", subcore_axis_name="s")
    return pl.kernel(
        functools.partial(_sc_gather_body, n_chunks),
        out_type=jax.ShapeDtypeStruct((n_rows * CHUNK, H2), jnp.uint32),
        mesh=mesh,
        scratch_types=[
            pltpu.VMEM((CHUNK,), jnp.int32),
            pltpu.VMEM((CHUNK,), jnp.int32),
            pltpu.VMEM((CHUNK, HIDDEN), jnp.float32),
            pltpu.VMEM((CHUNK, HIDDEN), jnp.float32),
            pltpu.VMEM((CHUNK, H2), jnp.uint32),
            pltpu.VMEM((CHUNK, H2), jnp.uint32),
            pltpu.SemaphoreType.DMA,
            pltpu.SemaphoreType.DMA,
            pltpu.SemaphoreType.DMA,
            pltpu.SemaphoreType.DMA,
        ],
    )(ids2d, table)


TOK_BLK = 512
N_BLKS = N_TOKENS // TOK_BLK


def _tc_ln_body(g_ref, pos_ref, tt_ref, type_ref, gam_ref, bet_ref, out_ref):
    w = g_ref[...]
    xl = lax.bitcast_convert_type(w << 16, jnp.float32)
    xh = lax.bitcast_convert_type(w & jnp.uint32(0xFFFF0000), jnp.float32)
    x = jnp.concatenate([xl, xh], axis=1) + pos_ref[...]
    t0 = type_ref[0:1, :]
    dt = type_ref[1:2, :] - t0
    tt = tt_ref[0, 0, :]
    x = x + t0 + tt[:, None] * dt
    mean = jnp.mean(x, axis=-1, keepdims=True)
    xc = x - mean
    var = jnp.mean(xc * xc, axis=-1, keepdims=True)
    y = xc * lax.rsqrt(var + EPS)
    out_ref[...] = y * gam_ref[...] + bet_ref[...]


SLICE_BATCHES = (6, 8, 9, 9)
SLICE_START = tuple(sum(SLICE_BATCHES[:s]) for s in range(len(SLICE_BATCHES)))


def _make_tc_ln(s):
    aliased = s > 0
    n_b = SLICE_BATCHES[s]

    def body(*refs):
        if aliased:
            g_ref, pos_ref, tt_ref, type_ref, gam_ref, bet_ref, _, out_ref = refs
        else:
            g_ref, pos_ref, tt_ref, type_ref, gam_ref, bet_ref, out_ref = refs
        _tc_ln_body(g_ref, pos_ref, tt_ref, type_ref, gam_ref, bet_ref,
                    out_ref)

    blk0 = SLICE_START[s]
    in_specs = [
        pl.BlockSpec((TOK_BLK, H2), lambda j: (j, 0)),
        pl.BlockSpec((TOK_BLK, HIDDEN), lambda j: (0, 0)),
        pl.BlockSpec((1, 1, TOK_BLK), lambda j: (j, 0, 0)),
        pl.BlockSpec((2, HIDDEN), lambda j: (0, 0)),
        pl.BlockSpec((1, HIDDEN), lambda j: (0, 0)),
        pl.BlockSpec((1, HIDDEN), lambda j: (0, 0)),
    ]
    if aliased:
        in_specs.append(pl.BlockSpec(memory_space=pl.ANY))
    return pl.pallas_call(
        body,
        grid=(n_b,),
        in_specs=in_specs,
        out_specs=pl.BlockSpec((TOK_BLK, HIDDEN), lambda j: (blk0 + j, 0)),
        out_shape=jax.ShapeDtypeStruct((N_TOKENS, HIDDEN), jnp.float32),
        input_output_aliases={6: 0} if aliased else {},
    )


@jax.jit
def _pipeline(ids2d, word_table, pos_table, ttf, type_table, gamma, beta):
    rows_per_batch = SEQ // CHUNK
    gs = [
        _sc_gather(
            lax.slice_in_dim(ids2d, SLICE_START[s] * rows_per_batch,
                             (SLICE_START[s] + SLICE_BATCHES[s]) * rows_per_batch,
                             axis=0),
            word_table)
        for s in range(len(SLICE_BATCHES))
    ]
    buf = None
    for s in range(len(SLICE_BATCHES)):
        tt_s = lax.slice_in_dim(ttf, SLICE_START[s],
                                SLICE_START[s] + SLICE_BATCHES[s], axis=0)
        args = (gs[s], pos_table, tt_s, type_table, gamma, beta)
        buf = _make_tc_ln(s)(*(args if buf is None else args + (buf,)))
    return buf


def kernel(input_ids, token_type_ids, word_table, pos_table, type_table,
           ln_gamma, ln_beta):
    ids2d = input_ids.reshape(N_TOKENS // CHUNK, CHUNK).astype(jnp.int32)
    ttf = token_type_ids.reshape(N_BLKS, 1, TOK_BLK).astype(jnp.float32)
    out = _pipeline(ids2d, word_table, pos_table, ttf, type_table,
                    ln_gamma.reshape(1, HIDDEN), ln_beta.reshape(1, HIDDEN))
    return out.reshape(BATCH, SEQ, HIDDEN)
